# Initial kernel scaffold; baseline (speedup 1.0000x reference)
import jax
import jax.numpy as jnp
from jax.experimental import pallas as pl
from jax.experimental.pallas import tpu as pltpu

N_EMBD = 768
NEURON_DIM = 512
TOP_K = 64
HIDDEN = 256
N_CHUNKS = 100000

BLK = 1024
NBLK = 98
NPAD = NBLK * BLK
NEG = jnp.float32(-3.0e38)


def _sims_kernel(ct_ref, bt_ref, out_ref):
    i = pl.program_id(0)
    ct = ct_ref[...]
    bt = bt_ref[...]
    d = jax.lax.dot_general(ct, bt, (((1,), (1,)), ((), ())),
                            preferred_element_type=jnp.float32)
    n2 = jnp.sum(ct * ct, axis=1, keepdims=True)
    row = jax.lax.broadcasted_iota(jnp.int32, (BLK, 1), 0) + i * BLK
    out_ref[...] = jnp.where(row < N_CHUNKS, d / (jnp.sqrt(n2) + 1e-8), NEG)


def _select_kernel(sims_ref, emb_hbm, ctr_hbm, be_ref, btr_ref,
                   w1_ref, b1_ref, w2_ref, b2_ref,
                   score_out, idx_out, sims_v, emb_s, tr_s, sem_s, sem_e, sem_t):
    pltpu.make_async_copy(sims_ref, sims_v, sem_s).start()
    pltpu.make_async_copy(sims_ref, sims_v, sem_s).wait()
    s = sims_v[...]
    nrow = NPAD // 128
    fr = jax.lax.broadcasted_iota(jnp.int32, (nrow, 128), 0)
    fc = jax.lax.broadcasted_iota(jnp.int32, (nrow, 128), 1)
    flat = fr * 128 + fc
    lane = jax.lax.broadcasted_iota(jnp.int32, (1, 128), 1)

    def body(k, carry):
        sv, ids = carry
        m = jnp.max(sv)
        pos = jnp.min(jnp.where(sv >= m, flat, jnp.int32(2 ** 30)))
        pltpu.make_async_copy(emb_hbm.at[pos], emb_s.at[k], sem_e).start()
        pltpu.make_async_copy(ctr_hbm.at[pos], tr_s.at[k], sem_t).start()
        ids = jnp.where(lane == k, pos, ids)
        sv = jnp.where(flat == pos, NEG, sv)
        return sv, ids

    ids0 = jnp.zeros((1, 128), jnp.int32)
    _, ids = jax.lax.fori_loop(0, TOP_K, body, (s, ids0))

    def wbody(k, c):
        pltpu.make_async_copy(emb_hbm.at[0], emb_s.at[0], sem_e).wait()
        pltpu.make_async_copy(ctr_hbm.at[0], tr_s.at[0], sem_t).wait()
        return c
    jax.lax.fori_loop(0, TOP_K, wbody, 0)

    e = emb_s[...]
    t = tr_s[...]
    w1a = w1_ref[0:N_EMBD, :]
    w1b = w1_ref[N_EMBD:2 * N_EMBD, :]
    w1c = w1_ref[2 * N_EMBD:2 * N_EMBD + NEURON_DIM, :]
    w1d = w1_ref[2 * N_EMBD + NEURON_DIM:, :]
    cvec = (jnp.dot(be_ref[...], w1b, preferred_element_type=jnp.float32)
            + jnp.dot(btr_ref[...], w1d, preferred_element_type=jnp.float32)
            + b1_ref[...])
    h = jnp.maximum(
        jnp.dot(e, w1a, preferred_element_type=jnp.float32)
        + jnp.dot(t, w1c, preferred_element_type=jnp.float32) + cvec, 0.0)
    scores = jnp.dot(h, w2_ref[...], preferred_element_type=jnp.float32) \
        + b2_ref[...]
    best = jnp.max(scores)
    r64 = jax.lax.broadcasted_iota(jnp.int32, (TOP_K, 1), 0)
    r = jnp.min(jnp.where(scores >= best, r64, jnp.int32(TOP_K)))
    cid = jnp.max(jnp.where(lane == r, ids, jnp.int32(-1)))
    score_out[0, 0] = best
    idx_out[0, 0] = cid


def kernel(backstory_embedding, backstory_trace, chunk_embeddings,
           chunk_traces, W1, b1, W2, b2):
    sims = pl.pallas_call(
        _sims_kernel,
        grid=(NBLK,),
        in_specs=[
            pl.BlockSpec((BLK, NEURON_DIM), lambda i: (i, 0)),
            pl.BlockSpec((1, NEURON_DIM), lambda i: (0, 0)),
        ],
        out_specs=pl.BlockSpec((BLK, 1), lambda i: (i, 0)),
        out_shape=jax.ShapeDtypeStruct((NPAD, 1), jnp.float32),
    )(chunk_traces, backstory_trace.reshape(1, NEURON_DIM))

    score, idx = pl.pallas_call(
        _select_kernel,
        in_specs=[
            pl.BlockSpec(memory_space=pltpu.ANY),
            pl.BlockSpec(memory_space=pltpu.ANY),
            pl.BlockSpec(memory_space=pltpu.ANY),
            pl.BlockSpec((1, N_EMBD), lambda: (0, 0)),
            pl.BlockSpec((1, NEURON_DIM), lambda: (0, 0)),
            pl.BlockSpec((2 * N_EMBD + 2 * NEURON_DIM, HIDDEN), lambda: (0, 0)),
            pl.BlockSpec((1, HIDDEN), lambda: (0, 0)),
            pl.BlockSpec((HIDDEN, 1), lambda: (0, 0)),
            pl.BlockSpec((1, 1), lambda: (0, 0)),
        ],
        out_specs=[
            pl.BlockSpec((1, 1), lambda: (0, 0)),
            pl.BlockSpec((1, 1), lambda: (0, 0)),
        ],
        out_shape=[
            jax.ShapeDtypeStruct((1, 1), jnp.float32),
            jax.ShapeDtypeStruct((1, 1), jnp.int32),
        ],
        scratch_shapes=[
            pltpu.VMEM((NPAD // 128, 128), jnp.float32),
            pltpu.VMEM((TOP_K, N_EMBD), jnp.float32),
            pltpu.VMEM((TOP_K, NEURON_DIM), jnp.float32),
            pltpu.SemaphoreType.DMA,
            pltpu.SemaphoreType.DMA,
            pltpu.SemaphoreType.DMA,
        ],
    )(sims.reshape(NPAD // 128, 128), chunk_embeddings, chunk_traces,
      backstory_embedding.reshape(1, N_EMBD),
      backstory_trace.reshape(1, NEURON_DIM),
      W1, b1.reshape(1, HIDDEN), W2, b2.reshape(1, 1))

    return score.reshape(()), idx.reshape(())


# speedup vs baseline: 1.5708x; 1.5708x over previous
import jax
import jax.numpy as jnp
from jax.experimental import pallas as pl
from jax.experimental.pallas import tpu as pltpu

N_EMBD = 768
NEURON_DIM = 512
TOP_K = 64
HIDDEN = 256
N_CHUNKS = 100000

BLK = 1024
NBLK = 98
NPAD = NBLK * BLK
NEG = -3.0e38


def _sims_kernel(ct_ref, bt_ref, out_ref):
    i = pl.program_id(0)
    ct = ct_ref[...]
    bt = bt_ref[...]
    d = jax.lax.dot_general(ct, bt, (((1,), (1,)), ((), ())),
                            preferred_element_type=jnp.float32)
    n2 = jnp.sum(ct * ct, axis=1, keepdims=True)
    row = jax.lax.broadcasted_iota(jnp.int32, (BLK, 1), 0) + i * BLK
    out_ref[...] = jnp.where(row < N_CHUNKS, d / (jnp.sqrt(n2) + 1e-8), NEG)


def _select_kernel(sims_ref, emb_hbm, ctr_hbm, be_ref, btr_ref,
                   w1_ref, b1_ref, w2_ref, b2_ref,
                   score_out, idx_out, sims_v, emb_s, tr_s, sem_s, sem_e, sem_t):
    pltpu.make_async_copy(sims_ref, sims_v, sem_s).start()
    pltpu.make_async_copy(sims_ref, sims_v, sem_s).wait()
    s = sims_v[...]
    nrow = NPAD // 128
    fr = jax.lax.broadcasted_iota(jnp.int32, (nrow, 128), 0)
    fc = jax.lax.broadcasted_iota(jnp.int32, (nrow, 128), 1)
    flat = fr * 128 + fc
    lane = jax.lax.broadcasted_iota(jnp.int32, (1, 128), 1)

    def body(k, carry):
        sv, ids = carry
        m = jnp.max(sv)
        pos = jnp.min(jnp.where(sv >= m, flat, jnp.int32(2 ** 30)))
        pltpu.make_async_copy(emb_hbm.at[pos], emb_s.at[k], sem_e).start()
        pltpu.make_async_copy(ctr_hbm.at[pos], tr_s.at[k], sem_t).start()
        ids = jnp.where(lane == k, pos, ids)
        sv = jnp.where(flat == pos, NEG, sv)
        return sv, ids

    ids0 = jnp.zeros((1, 128), jnp.int32)
    _, ids = jax.lax.fori_loop(0, TOP_K, body, (s, ids0))

    def wbody(k, c):
        pltpu.make_async_copy(emb_hbm.at[0], emb_s.at[0], sem_e).wait()
        pltpu.make_async_copy(ctr_hbm.at[0], tr_s.at[0], sem_t).wait()
        return c
    jax.lax.fori_loop(0, TOP_K, wbody, 0)

    e = emb_s[...]
    t = tr_s[...]
    w1a = w1_ref[0:N_EMBD, :]
    w1b = w1_ref[N_EMBD:2 * N_EMBD, :]
    w1c = w1_ref[2 * N_EMBD:2 * N_EMBD + NEURON_DIM, :]
    w1d = w1_ref[2 * N_EMBD + NEURON_DIM:, :]
    cvec = (jnp.dot(be_ref[...], w1b, preferred_element_type=jnp.float32)
            + jnp.dot(btr_ref[...], w1d, preferred_element_type=jnp.float32)
            + b1_ref[...])
    h = jnp.maximum(
        jnp.dot(e, w1a, preferred_element_type=jnp.float32)
        + jnp.dot(t, w1c, preferred_element_type=jnp.float32) + cvec, 0.0)
    scores = jnp.dot(h, w2_ref[...], preferred_element_type=jnp.float32) \
        + b2_ref[...]
    best = jnp.max(scores)
    r64 = jax.lax.broadcasted_iota(jnp.int32, (TOP_K, 1), 0)
    r = jnp.min(jnp.where(scores >= best, r64, jnp.int32(TOP_K)))
    cid = jnp.max(jnp.where(lane == r, ids, jnp.int32(-1)))
    score_out[0, 0] = best
    idx_out[0, 0] = cid


def kernel(backstory_embedding, backstory_trace, chunk_embeddings,
           chunk_traces, W1, b1, W2, b2):
    sims = pl.pallas_call(
        _sims_kernel,
        grid=(NBLK,),
        in_specs=[
            pl.BlockSpec((BLK, NEURON_DIM), lambda i: (i, 0)),
            pl.BlockSpec((1, NEURON_DIM), lambda i: (0, 0)),
        ],
        out_specs=pl.BlockSpec((BLK, 1), lambda i: (i, 0)),
        out_shape=jax.ShapeDtypeStruct((NPAD, 1), jnp.float32),
    )(chunk_traces, backstory_trace.reshape(1, NEURON_DIM))

    score, idx = pl.pallas_call(
        _select_kernel,
        in_specs=[
            pl.BlockSpec(memory_space=pl.ANY),
            pl.BlockSpec(memory_space=pl.ANY),
            pl.BlockSpec(memory_space=pl.ANY),
            pl.BlockSpec((1, N_EMBD), lambda: (0, 0)),
            pl.BlockSpec((1, NEURON_DIM), lambda: (0, 0)),
            pl.BlockSpec((2 * N_EMBD + 2 * NEURON_DIM, HIDDEN), lambda: (0, 0)),
            pl.BlockSpec((1, HIDDEN), lambda: (0, 0)),
            pl.BlockSpec((HIDDEN, 1), lambda: (0, 0)),
            pl.BlockSpec((1, 1), lambda: (0, 0)),
        ],
        out_specs=[
            pl.BlockSpec(memory_space=pltpu.SMEM),
            pl.BlockSpec(memory_space=pltpu.SMEM),
        ],
        out_shape=[
            jax.ShapeDtypeStruct((1, 1), jnp.float32),
            jax.ShapeDtypeStruct((1, 1), jnp.int32),
        ],
        scratch_shapes=[
            pltpu.VMEM((NPAD // 128, 128), jnp.float32),
            pltpu.VMEM((TOP_K, N_EMBD), jnp.float32),
            pltpu.VMEM((TOP_K, NEURON_DIM), jnp.float32),
            pltpu.SemaphoreType.DMA,
            pltpu.SemaphoreType.DMA,
            pltpu.SemaphoreType.DMA,
        ],
    )(sims.reshape(NPAD // 128, 128), chunk_embeddings, chunk_traces,
      backstory_embedding.reshape(1, N_EMBD),
      backstory_trace.reshape(1, NEURON_DIM),
      W1, b1.reshape(1, HIDDEN), W2, b2.reshape(1, 1))

    return score.reshape(()), idx.reshape(())
